# baseline (speedup 1.0000x reference)
import jax
import jax.numpy as jnp
from jax.experimental import pallas as pl
from jax.experimental.pallas import tpu as pltpu

B, T, D = 4, 8192, 768
E = 8
TOP_K = 2
CITY_DIM = 32

BLK = 2048

NEG_INF = float("-inf")


def _router_body(mh_ref, dt_ref, dd_ref, rg_ref, de_ref, w_ref, crow_ref,
                 b_ref, router_ref, idx_ref, gate_ref):
    acc = jnp.dot(mh_ref[0], jnp.transpose(w_ref[:, 0:768]),
                  preferred_element_type=jnp.float32)
    lt = jnp.transpose(acc)
    j = pl.program_id(1)
    cols = pl.ds(j * BLK, BLK)
    lt += jnp.dot(w_ref[:, 800:992], dt_ref[0, :, cols],
                  preferred_element_type=jnp.float32)
    lt += jnp.dot(w_ref[:, 992:1184], dd_ref[0, :, cols],
                  preferred_element_type=jnp.float32)
    lt += jnp.dot(w_ref[:, 1184:1280], rg_ref[0, :, cols],
                  preferred_element_type=jnp.float32)
    lt += jnp.dot(w_ref[:, 1280:1376], de_ref[0, :, cols],
                  preferred_element_type=jnp.float32)
    cb = jnp.dot(w_ref[:, 768:800], crow_ref[...],
                 preferred_element_type=jnp.float32)
    lt += cb + b_ref[...]

    srow = jax.lax.broadcasted_iota(jnp.int32, lt.shape, 0).astype(jnp.float32)

    m1 = jnp.max(lt, axis=0, keepdims=True)
    ex = jnp.exp(lt - m1)
    gate_ref[...] = (ex / jnp.sum(ex, axis=0, keepdims=True))[None]

    idx1 = jnp.min(jnp.where(lt == m1, srow, float(E)), axis=0, keepdims=True)
    is1 = srow == idx1
    l2 = jnp.where(is1, NEG_INF, lt)
    m2 = jnp.max(l2, axis=0, keepdims=True)
    idx2 = jnp.min(jnp.where(l2 == m2, srow, float(E)), axis=0, keepdims=True)
    is2 = srow == idx2

    e2 = jnp.exp(m2 - m1)
    denom = 1.0 + e2
    p1 = jnp.broadcast_to(1.0 / denom, lt.shape)
    p2 = jnp.broadcast_to(e2 / denom, lt.shape)
    router_ref[...] = jnp.where(is1, p1, jnp.where(is2, p2, 0.0))[None]

    idx_ref[...] = jnp.concatenate([idx1, idx2], axis=0).astype(jnp.int32)[None]


@jax.jit
def _run(mh, dtT, ddT, rgT, deT, w, crow, b):
    grid = (B, T // BLK)
    tok = lambda b, i: (b, i, 0)
    feat = lambda b, i: (b, 0, i)
    fixed = lambda b, i: (0, 0)
    out = pl.pallas_call(
        _router_body,
        grid=grid,
        in_specs=[
            pl.BlockSpec((1, BLK, D), tok),
            pl.BlockSpec((1, D // 4, T), lambda b, i: (b, 0, 0)),
            pl.BlockSpec((1, D // 4, T), lambda b, i: (b, 0, 0)),
            pl.BlockSpec((1, D // 8, T), lambda b, i: (b, 0, 0)),
            pl.BlockSpec((1, D // 8, T), lambda b, i: (b, 0, 0)),
            pl.BlockSpec(w.shape, fixed),
            pl.BlockSpec(crow.shape, fixed),
            pl.BlockSpec(b.shape, fixed),
        ],
        out_specs=[
            pl.BlockSpec((1, E, BLK), feat),
            pl.BlockSpec((1, TOP_K, BLK), feat),
            pl.BlockSpec((1, E, BLK), feat),
        ],
        out_shape=[
            jax.ShapeDtypeStruct((B, E, T), jnp.float32),
            jax.ShapeDtypeStruct((B, TOP_K, T), jnp.int32),
            jax.ShapeDtypeStruct((B, E, T), jnp.float32),
        ],
        compiler_params=pltpu.CompilerParams(
            dimension_semantics=("arbitrary", "arbitrary"),
        ),
    )(mh, dtT, ddT, rgT, deT, w, crow, b)
    return out


def kernel(mh_output, delta_t_info, delta_dis_info, delta_rg_info,
           delta_entropy_info, city_embeddings, W_topk, b_topk, city):
    crow = city_embeddings[city].reshape(CITY_DIM, 1)
    swap = lambda a: jnp.transpose(a, (0, 2, 1))
    routerT, idxT, gateT = _run(
        mh_output, swap(delta_t_info), swap(delta_dis_info),
        swap(delta_rg_info), swap(delta_entropy_info), jnp.transpose(W_topk),
        crow, b_topk.reshape(E, 1))
    return (swap(routerT), swap(idxT), swap(gateT))


# speedup vs baseline: 1.0871x; 1.0871x over previous
import jax
import jax.numpy as jnp
from jax.experimental import pallas as pl
from jax.experimental.pallas import tpu as pltpu

B, T, D = 4, 8192, 768
E = 8
TOP_K = 2
CITY_DIM = 32

BLK = 2048

NEG_INF = float("-inf")


def _router_body(mh_ref, dt_ref, dd_ref, rg_ref, de_ref, w_ref, crow_ref,
                 b_ref, router_ref, idx_ref, gate_ref):
    acc = jnp.dot(mh_ref[0], jnp.transpose(w_ref[:, 0:768]),
                  preferred_element_type=jnp.float32)
    lt = jnp.transpose(acc)
    lt += jnp.dot(w_ref[:, 800:992], dt_ref[0],
                  preferred_element_type=jnp.float32)
    lt += jnp.dot(w_ref[:, 992:1184], dd_ref[0],
                  preferred_element_type=jnp.float32)
    lt += jnp.dot(w_ref[:, 1184:1280], rg_ref[0],
                  preferred_element_type=jnp.float32)
    lt += jnp.dot(w_ref[:, 1280:1376], de_ref[0],
                  preferred_element_type=jnp.float32)
    cb = jnp.dot(w_ref[:, 768:800], crow_ref[...],
                 preferred_element_type=jnp.float32)
    lt += cb + b_ref[...]

    srow = jax.lax.broadcasted_iota(jnp.int32, lt.shape, 0).astype(jnp.float32)

    m1 = jnp.max(lt, axis=0, keepdims=True)
    ex = jnp.exp(lt - m1)
    gate_ref[...] = (ex / jnp.sum(ex, axis=0, keepdims=True))[None]

    idx1 = jnp.min(jnp.where(lt == m1, srow, float(E)), axis=0, keepdims=True)
    is1 = srow == idx1
    l2 = jnp.where(is1, NEG_INF, lt)
    m2 = jnp.max(l2, axis=0, keepdims=True)
    idx2 = jnp.min(jnp.where(l2 == m2, srow, float(E)), axis=0, keepdims=True)
    is2 = srow == idx2

    e2 = jnp.exp(m2 - m1)
    denom = 1.0 + e2
    p1 = jnp.broadcast_to(1.0 / denom, lt.shape)
    p2 = jnp.broadcast_to(e2 / denom, lt.shape)
    router_ref[...] = jnp.where(is1, p1, jnp.where(is2, p2, 0.0))[None]

    idx_ref[...] = jnp.concatenate([idx1, idx2], axis=0).astype(jnp.int32)[None]


@jax.jit
def _run(mh, dtT, ddT, rgT, deT, w, crow, b):
    grid = (B, T // BLK)
    tok = lambda b, i: (b, i, 0)
    feat = lambda b, i: (b, 0, i)
    fixed = lambda b, i: (0, 0)
    out = pl.pallas_call(
        _router_body,
        grid=grid,
        in_specs=[
            pl.BlockSpec((1, BLK, D), tok),
            pl.BlockSpec((1, D // 4, BLK), feat),
            pl.BlockSpec((1, D // 4, BLK), feat),
            pl.BlockSpec((1, D // 8, BLK), feat),
            pl.BlockSpec((1, D // 8, BLK), feat),
            pl.BlockSpec(w.shape, fixed),
            pl.BlockSpec(crow.shape, fixed),
            pl.BlockSpec(b.shape, fixed),
        ],
        out_specs=[
            pl.BlockSpec((1, E, BLK), feat),
            pl.BlockSpec((1, TOP_K, BLK), feat),
            pl.BlockSpec((1, E, BLK), feat),
        ],
        out_shape=[
            jax.ShapeDtypeStruct((B, E, T), jnp.float32),
            jax.ShapeDtypeStruct((B, TOP_K, T), jnp.int32),
            jax.ShapeDtypeStruct((B, E, T), jnp.float32),
        ],
        compiler_params=pltpu.CompilerParams(
            dimension_semantics=("arbitrary", "arbitrary"),
        ),
    )(mh, dtT, ddT, rgT, deT, w, crow, b)
    return out


def kernel(mh_output, delta_t_info, delta_dis_info, delta_rg_info,
           delta_entropy_info, city_embeddings, W_topk, b_topk, city):
    crow = city_embeddings[city].reshape(CITY_DIM, 1)
    swap = lambda a: jnp.transpose(a, (0, 2, 1))
    routerT, idxT, gateT = _run(
        mh_output, swap(delta_t_info), swap(delta_dis_info),
        swap(delta_rg_info), swap(delta_entropy_info), jnp.transpose(W_topk),
        crow, b_topk.reshape(E, 1))
    return (swap(routerT), swap(idxT), swap(gateT))
